# Initial kernel scaffold; baseline (speedup 1.0000x reference)
import jax
import jax.numpy as jnp
from jax.experimental import pallas as pl


def kernel(posteriors, W1, b1, g1, be1, W2, b2, g2, be2, W3, b3):
    raise NotImplementedError("write your pallas kernel here")


# speedup vs baseline: 12.4059x; 12.4059x over previous
import functools

import jax
import jax.numpy as jnp
from jax.experimental import pallas as pl

_B, _E, _C = 4096, 8, 1000
_EPS = 1e-08
_TB = 128


def _body(p_ref, W1_ref, b1_ref, g1_ref, be1_ref, W2_ref, b2_ref, g2_ref,
          be2_ref, W3_ref, b3_ref, out_ref):
    p = p_ref[...]
    lp = jnp.log(p + _EPS)
    ent = -jnp.sum(p * lp, axis=-1)
    m = jnp.mean(p, axis=1)
    lm = jnp.log(m + _EPS)
    plm = jnp.sum(p * lm[:, None, :], axis=-1)
    kl = -ent - plm
    pm = jnp.sum(p * m[:, None, :], axis=-1)
    p2 = jnp.sum(p * p, axis=-1)
    m2 = jnp.sum(m * m, axis=-1)
    pn = jnp.sqrt(p2)
    mn = jnp.sqrt(m2)
    cos = pm / (jnp.maximum(pn, _EPS) * jnp.maximum(mn, _EPS)[:, None])

    x = p
    v1 = jnp.max(x, axis=-1)
    x = jnp.where(x == v1[..., None], -1.0, x)
    v2 = jnp.max(x, axis=-1)
    x = jnp.where(x == v2[..., None], -1.0, x)
    v3 = jnp.max(x, axis=-1)
    x = jnp.where(x == v3[..., None], -1.0, x)
    v4 = jnp.max(x, axis=-1)
    x = jnp.where(x == v4[..., None], -1.0, x)
    v5 = jnp.max(x, axis=-1)
    mp = v1
    tm = v1 + v2 + v3 + v4 + v5
    rm = 1.0 - tm
    gap = v1 - v2

    ment = -jnp.sum(m * lm, axis=-1)
    d = p - m[:, None, :]
    mcv = jnp.mean(jnp.sum(d * d, axis=1) / 7.0, axis=-1)
    mu_mp = jnp.mean(mp, axis=-1, keepdims=True)
    smc = jnp.sqrt(jnp.sum((mp - mu_mp) ** 2, axis=-1) / 7.0)

    gl = jnp.concatenate([ment[:, None], mcv[:, None], smc[:, None]], axis=1)
    f = jnp.concatenate([ent, tm, rm, mp, gap, cos, kl, gl], axis=1)
    f = jnp.clip(f, -100.0, 100.0)

    def ln(x, g, b):
        mu = jnp.mean(x, axis=-1, keepdims=True)
        v = jnp.mean((x - mu) ** 2, axis=-1, keepdims=True)
        return (x - mu) / jnp.sqrt(v + 1e-5) * g + b

    h = jnp.dot(f, W1_ref[...], preferred_element_type=jnp.float32) + b1_ref[...]
    h = jax.nn.relu(ln(h, g1_ref[...], be1_ref[...]))
    h = jnp.dot(h, W2_ref[...], preferred_element_type=jnp.float32) + b2_ref[...]
    h = jax.nn.relu(ln(h, g2_ref[...], be2_ref[...]))
    logits = jnp.dot(h, W3_ref[...], preferred_element_type=jnp.float32) + b3_ref[...]
    z = logits - jnp.max(logits, axis=-1, keepdims=True)
    ez = jnp.exp(z)
    out_ref[...] = ez / jnp.sum(ez, axis=-1, keepdims=True)


@jax.jit
def kernel(posteriors, W1, b1, g1, be1, W2, b2, g2, be2, W3, b3):
    grid = (_B // _TB,)
    full = lambda shape: pl.BlockSpec(shape, lambda i: (0,) * len(shape))
    return pl.pallas_call(
        _body,
        grid=grid,
        in_specs=[
            pl.BlockSpec((_TB, _E, _C), lambda i: (i, 0, 0)),
            full((59, 256)), full((1, 256)), full((1, 256)), full((1, 256)),
            full((256, 128)), full((1, 128)), full((1, 128)), full((1, 128)),
            full((128, _E)), full((1, _E)),
        ],
        out_specs=pl.BlockSpec((_TB, _E), lambda i: (i, 0)),
        out_shape=jax.ShapeDtypeStruct((_B, _E), jnp.float32),
    )(posteriors, W1, b1.reshape(1, -1), g1.reshape(1, -1), be1.reshape(1, -1),
      W2, b2.reshape(1, -1), g2.reshape(1, -1), be2.reshape(1, -1),
      W3, b3.reshape(1, -1))


# speedup vs baseline: 14.2399x; 1.1478x over previous
import functools

import jax
import jax.numpy as jnp
from jax.experimental import pallas as pl

_B, _E, _C = 4096, 8, 1000
_EPS = 1e-08
_TB = 128


def _body(p_ref, W1_ref, b1_ref, g1_ref, be1_ref, W2_ref, b2_ref, g2_ref,
          be2_ref, W3_ref, b3_ref, out_ref):
    p = p_ref[...]
    lp = jnp.log(p + _EPS)
    ent = -jnp.sum(p * lp, axis=-1)
    m = jnp.mean(p, axis=1)
    lm = jnp.log(m + _EPS)
    plm = jnp.sum(p * lm[:, None, :], axis=-1)
    kl = -ent - plm
    pm = jnp.sum(p * m[:, None, :], axis=-1)
    p2 = jnp.sum(p * p, axis=-1)
    m2 = jnp.sum(m * m, axis=-1)
    pn = jnp.sqrt(p2)
    mn = jnp.sqrt(m2)
    cos = pm / (jnp.maximum(pn, _EPS) * jnp.maximum(mn, _EPS)[:, None])

    x = p
    v1 = jnp.max(x, axis=-1)
    x = jnp.where(x == v1[..., None], -1.0, x)
    v2 = jnp.max(x, axis=-1)
    x = jnp.where(x == v2[..., None], -1.0, x)
    v3 = jnp.max(x, axis=-1)
    x = jnp.where(x == v3[..., None], -1.0, x)
    v4 = jnp.max(x, axis=-1)
    x = jnp.where(x == v4[..., None], -1.0, x)
    v5 = jnp.max(x, axis=-1)
    mp = v1
    tm = v1 + v2 + v3 + v4 + v5
    rm = 1.0 - tm
    gap = v1 - v2

    ment = -jnp.sum(m * lm, axis=-1)
    mcv = (jnp.sum(p2, axis=-1) - 8.0 * m2) / 7000.0
    mu_mp = jnp.mean(mp, axis=-1, keepdims=True)
    smc = jnp.sqrt(jnp.sum((mp - mu_mp) ** 2, axis=-1) / 7.0)

    gl = jnp.concatenate([ment[:, None], mcv[:, None], smc[:, None]], axis=1)
    f = jnp.concatenate([ent, tm, rm, mp, gap, cos, kl, gl], axis=1)
    f = jnp.clip(f, -100.0, 100.0)

    def ln(x, g, b):
        mu = jnp.mean(x, axis=-1, keepdims=True)
        v = jnp.mean((x - mu) ** 2, axis=-1, keepdims=True)
        return (x - mu) / jnp.sqrt(v + 1e-5) * g + b

    h = jnp.dot(f, W1_ref[...], preferred_element_type=jnp.float32) + b1_ref[...]
    h = jax.nn.relu(ln(h, g1_ref[...], be1_ref[...]))
    h = jnp.dot(h, W2_ref[...], preferred_element_type=jnp.float32) + b2_ref[...]
    h = jax.nn.relu(ln(h, g2_ref[...], be2_ref[...]))
    logits = jnp.dot(h, W3_ref[...], preferred_element_type=jnp.float32) + b3_ref[...]
    z = logits - jnp.max(logits, axis=-1, keepdims=True)
    ez = jnp.exp(z)
    out_ref[...] = ez / jnp.sum(ez, axis=-1, keepdims=True)


@jax.jit
def kernel(posteriors, W1, b1, g1, be1, W2, b2, g2, be2, W3, b3):
    grid = (_B // _TB,)
    full = lambda shape: pl.BlockSpec(shape, lambda i: (0,) * len(shape))
    return pl.pallas_call(
        _body,
        grid=grid,
        in_specs=[
            pl.BlockSpec((_TB, _E, _C), lambda i: (i, 0, 0)),
            full((59, 256)), full((1, 256)), full((1, 256)), full((1, 256)),
            full((256, 128)), full((1, 128)), full((1, 128)), full((1, 128)),
            full((128, _E)), full((1, _E)),
        ],
        out_specs=pl.BlockSpec((_TB, _E), lambda i: (i, 0)),
        out_shape=jax.ShapeDtypeStruct((_B, _E), jnp.float32),
    )(posteriors, W1, b1.reshape(1, -1), g1.reshape(1, -1), be1.reshape(1, -1),
      W2, b2.reshape(1, -1), g2.reshape(1, -1), be2.reshape(1, -1),
      W3, b3.reshape(1, -1))
